# baseline (speedup 1.0000x reference)
import jax
import jax.numpy as jnp
from jax.experimental import pallas as pl
from jax.experimental.pallas import tpu as pltpu


def _fused_kernel(x_ref, adj_ref, w_ref, out_ref,
                  adjn_sc, m0_acc, cs_acc, mn_sc):
    p = pl.program_id(0)
    i = pl.program_id(1)
    num_tiles = pl.num_programs(1)
    tile = adj_ref.shape[0]
    a = adj_ref.shape[1]

    @pl.when(jnp.logical_and(p == 0, i == 0))
    def _init():
        m0_acc[...] = jnp.zeros_like(m0_acc)
        cs_acc[...] = jnp.zeros_like(cs_acc)

    @pl.when(p == 0)
    def _phase0():
        adj = adj_ref[...]
        x = x_ref[...]
        adj_bf = adj.astype(jnp.bfloat16)
        m0_acc[...] += jax.lax.dot_general(
            adj_bf, x.astype(jnp.bfloat16),
            (((0,), (0,)), ((), ())), preferred_element_type=jnp.float32)
        cs_acc[...] += jnp.sum(adj, axis=0, keepdims=True)
        ones_bf = jnp.ones((a, a), dtype=jnp.bfloat16)
        rsb = jax.lax.dot_general(
            adj_bf, ones_bf, (((1,), (0,)), ((), ())),
            preferred_element_type=jnp.float32)
        adjn_sc[pl.ds(i * tile, tile), :] = (adj / (rsb + 1e-12)).astype(jnp.bfloat16)

        @pl.when(i == num_tiles - 1)
        def _finish():
            rcol = 1.0 / (cs_acc[...] + 1e-12)
            row_id = jax.lax.broadcasted_iota(jnp.int32, (a, a), 0)
            col_id = jax.lax.broadcasted_iota(jnp.int32, (a, a), 1)
            dm = jnp.where(row_id == col_id, rcol, 0.0)
            m0n = jax.lax.dot_general(
                dm.astype(jnp.bfloat16), m0_acc[...].astype(jnp.bfloat16),
                (((1,), (0,)), ((), ())), preferred_element_type=jnp.float32)
            mn = jax.lax.dot_general(
                m0n.astype(jnp.bfloat16), w_ref[...].astype(jnp.bfloat16),
                (((1,), (0,)), ((), ())), preferred_element_type=jnp.float32)
            mn_sc[...] = mn.astype(jnp.bfloat16)

    @pl.when(p == 1)
    def _phase1():
        adjn = adjn_sc[pl.ds(i * tile, tile), :]
        out_ref[...] = jax.lax.dot_general(
            adjn, mn_sc[...], (((1,), (0,)), ((), ())),
            preferred_element_type=jnp.float32)


def _pick_tile(n):
    for t in (10000, 5000, 4000, 2500, 2000, 1000, 500, 200, 100, 40, 8):
        if n % t == 0 and t % 8 == 0:
            return t
    return n


def kernel(input, adj, W, anchor_mp):
    n, d_in = input.shape
    a = adj.shape[1]
    d_out = W.shape[1]
    tile = _pick_tile(n)
    num_tiles = n // tile

    w_scaled = W * jnp.asarray(anchor_mp, W.dtype)

    out = pl.pallas_call(
        _fused_kernel,
        grid=(2, num_tiles),
        in_specs=[
            pl.BlockSpec((tile, d_in), lambda p, i: (i * (1 - p), 0)),
            pl.BlockSpec((tile, a), lambda p, i: (i * (1 - p), 0)),
            pl.BlockSpec((d_in, d_out), lambda p, i: (0, 0)),
        ],
        out_specs=pl.BlockSpec((tile, d_out), lambda p, i: (i * p, 0)),
        out_shape=jax.ShapeDtypeStruct((n, d_out), jnp.float32),
        scratch_shapes=[
            pltpu.VMEM((n, a), jnp.bfloat16),
            pltpu.VMEM((a, d_in), jnp.float32),
            pltpu.VMEM((1, a), jnp.float32),
            pltpu.VMEM((a, d_out), jnp.bfloat16),
        ],
    )(input, adj, w_scaled)
    return out
